# baseline (speedup 1.0000x reference)
import functools

import jax
import jax.numpy as jnp
from jax import lax
from jax.experimental import pallas as pl
from jax.experimental.pallas import tpu as pltpu
from jax.experimental.pallas import tpu_sc as plsc

NC = 2
NS = 16
LANES = 16
K = 32
G = 16
R = 4
S = 2
DH = 256
DHW = DH // 2


@functools.lru_cache(maxsize=None)
def _make_diffusion(n, ec):
  half = n // 2
  n_per_tile = ec // NS
  n_chunks = n_per_tile // K
  n_groups = n_chunks // G
  cpr = 128 // K
  grows = G * K // 128
  assert n_chunks % R == 0 and n_chunks % G == 0 and n_groups >= 2
  assert G >= 2 * R
  rows_per_tile = (half // NS) // 8 * 8
  rows_rem = half - NS * rows_per_tile
  mesh = plsc.VectorSubcoreMesh(core_axis_name="c", subcore_axis_name="s")

  @functools.partial(
      pl.kernel,
      mesh=mesh,
      out_type=jax.ShapeDtypeStruct((n, 2, DHW), jnp.float32),
      scratch_types=(
          [
              pltpu.VMEM((2, grows, 128), jnp.int32),
              pltpu.VMEM((2, grows, 128), jnp.int32),
              pltpu.VMEM((2, grows, 128), jnp.float32),
          ]
          + [pltpu.VMEM((K, DHW), jnp.int32) for _ in range(R)]
          + [pltpu.VMEM((K, 2, DHW), jnp.float32) for _ in range(S)]
          + [pltpu.VMEM((K,), jnp.int32) for _ in range(S)]
          + [pltpu.VMEM_SHARED((half, 2, DHW), jnp.float32)]
          + [pltpu.SemaphoreType.DMA for _ in range(R + S + 1)]
      ),
  )
  def diffuse(xp_hbm, src_hbm, dst_hbm, w_hbm, z_hbm, out_hbm,
              sidx, didx, wts, *rest):
    rows_g = rest[0:R]
    rows_s = rest[R:R + S]
    dbuf = rest[R + S:R + 2 * S]
    acc = rest[R + 2 * S]
    semg = rest[R + 2 * S + 1:2 * R + 2 * S + 1]
    sems = rest[2 * R + 2 * S + 1:2 * R + 3 * S + 1]
    semt = rest[2 * R + 3 * S + 1]
    c = lax.axis_index("c")
    s = lax.axis_index("s")

    def issue_tables(grp, slot):
      sl = pl.ds(grp * grows, grows)
      pltpu.async_copy(src_hbm.at[c, s, sl], sidx.at[slot], semt)
      pltpu.async_copy(dst_hbm.at[c, s, sl], didx.at[slot], semt)
      pltpu.async_copy(w_hbm.at[c, s, sl], wts.at[slot], semt)

    def wait_tables(grp, slot):
      sl = pl.ds(grp * grows, grows)
      pltpu.make_async_copy(src_hbm.at[c, s, sl], sidx.at[slot], semt).wait()
      pltpu.make_async_copy(dst_hbm.at[c, s, sl], didx.at[slot], semt).wait()
      pltpu.make_async_copy(w_hbm.at[c, s, sl], wts.at[slot], semt).wait()

    r0 = s * rows_per_tile
    pltpu.sync_copy(z_hbm.at[pl.ds(r0, rows_per_tile)],
                    acc.at[pl.ds(r0, rows_per_tile)])
    if rows_rem:
      @pl.when(s == NS - 1)
      def _zero_tail():
        rt = NS * rows_per_tile
        pltpu.sync_copy(z_hbm.at[pl.ds(rt, rows_rem)],
                        acc.at[pl.ds(rt, rows_rem)])

    issue_tables(0, 0)
    wait_tables(0, 0)
    plsc.subcore_barrier()

    def gidx(t):
      slot = (t // G) % 2
      tt = t % G
      return sidx.at[slot, tt // cpr, pl.ds((tt % cpr) * K, K)]

    for b in range(R):
      pltpu.async_copy(xp_hbm.at[gidx(b)], rows_g[b], semg[b])

    def ring_body(p, carry):
      for b in range(R):
        t = p * R + b
        bs = b % S
        slot = (t // G) % 2
        tt = t % G
        pltpu.make_async_copy(xp_hbm.at[gidx(t)], rows_g[b], semg[b]).wait()
        def _drain():
          pltpu.make_async_copy(rows_s[bs], acc.at[dbuf[bs]],
                                sems[bs]).wait()
        if b >= S:
          _drain()
        else:
          pl.when(p > 0)(_drain)

        dbuf[bs][pl.ds(0, K)] = didx[slot, tt // cpr,
                                     pl.ds((tt % cpr) * K, K)]

        def edge_body(g16, c2):
          w16 = wts[slot, tt // cpr, pl.ds((tt % cpr) * K + g16 * LANES,
                                           LANES)]
          for l in range(LANES):
            i = g16 * LANES + l
            wb = w16[l]
            for j in range(DHW // LANES):
              fsl = pl.ds(j * LANES, LANES)
              v = rows_g[b][i, fsl]
              lo = lax.bitcast_convert_type(v << 16, jnp.float32)
              hi = lax.bitcast_convert_type(v & jnp.int32(-65536),
                                            jnp.float32)
              rows_s[bs][i, 0, fsl] = lo * wb
              rows_s[bs][i, 1, fsl] = hi * wb
          return c2

        lax.fori_loop(0, K // LANES, edge_body, 0)

        pltpu.async_copy(rows_s[bs], acc.at[dbuf[bs]], sems[bs], add=True)

        @pl.when((tt == R - 1) & (t < (n_groups - 1) * G))
        def _refill():
          issue_tables(t // G + 1, 1 - slot)

        t2 = t + R
        @pl.when((t2 % G == 0) & (t2 < n_chunks))
        def _tables_ready():
          wait_tables(t2 // G, (t2 // G) % 2)

        @pl.when(t2 < n_chunks)
        def _prefetch():
          pltpu.async_copy(xp_hbm.at[gidx(t2)], rows_g[b], semg[b])
      return carry

    lax.fori_loop(0, n_chunks // R, ring_body, 0)
    for bs in range(S):
      pltpu.make_async_copy(rows_s[bs], acc.at[dbuf[bs]], sems[bs]).wait()
    plsc.subcore_barrier()

    pltpu.sync_copy(acc.at[pl.ds(r0, rows_per_tile)],
                    out_hbm.at[pl.ds(c * half + r0, rows_per_tile)])
    if rows_rem:
      @pl.when(s == NS - 1)
      def _out_tail():
        rt = NS * rows_per_tile
        pltpu.sync_copy(acc.at[pl.ds(rt, rows_rem)],
                        out_hbm.at[pl.ds(c * half + rt, rows_rem)])

  return diffuse


def _pack_rows(y):
  lo = y[:, :DHW]
  hi = y[:, DHW:]
  lo_u = lax.bitcast_convert_type(lo.astype(jnp.bfloat16),
                                  jnp.uint16).astype(jnp.uint32)
  hi_u = lax.bitcast_convert_type(hi.astype(jnp.bfloat16),
                                  jnp.uint16).astype(jnp.uint32)
  return lax.bitcast_convert_type(lo_u | (hi_u << 16), jnp.int32)


def _mm_in(h, w_in):
  n, d_in = h.shape
  bn = 1000

  def body(h_ref, w_ref, o_ref, p_ref):
    y = jnp.dot(h_ref[...], w_ref[...], preferred_element_type=jnp.float32)
    o_ref[...] = y
    p_ref[...] = _pack_rows(y)

  return pl.pallas_call(
      body,
      grid=(n // bn,),
      in_specs=[
          pl.BlockSpec((bn, d_in), lambda i: (i, 0)),
          pl.BlockSpec((d_in, DH), lambda i: (0, 0)),
      ],
      out_specs=[
          pl.BlockSpec((bn, DH), lambda i: (i, 0)),
          pl.BlockSpec((bn, DHW), lambda i: (i, 0)),
      ],
      out_shape=[
          jax.ShapeDtypeStruct((n, DH), jnp.float32),
          jax.ShapeDtypeStruct((n, DHW), jnp.int32),
      ],
  )(h, w_in)


def _mm_layer(agg, x, w):
  n = x.shape[0]
  bn = 1000

  def body(a_ref, x_ref, w_ref, o_ref, p_ref):
    y = jnp.dot(a_ref[...], w_ref[...], preferred_element_type=jnp.float32)
    y = y * jax.nn.sigmoid(y) + x_ref[...]
    o_ref[...] = y
    p_ref[...] = _pack_rows(y)

  spec = pl.BlockSpec((bn, DH), lambda i: (i, 0))
  return pl.pallas_call(
      body,
      grid=(n // bn,),
      in_specs=[spec, spec, pl.BlockSpec((DH, DH), lambda i: (0, 0))],
      out_specs=[spec, pl.BlockSpec((bn, DHW), lambda i: (i, 0))],
      out_shape=[
          jax.ShapeDtypeStruct((n, DH), jnp.float32),
          jax.ShapeDtypeStruct((n, DHW), jnp.int32),
      ],
  )(agg, x, w)


def _mm_out(x, w_out):
  n = x.shape[0]
  d_out = w_out.shape[1]
  bn = 1000

  def body(x_ref, w_ref, o_ref):
    o_ref[...] = jnp.dot(x_ref[...], w_ref[...],
                         preferred_element_type=jnp.float32)

  return pl.pallas_call(
      body,
      grid=(n // bn,),
      in_specs=[
          pl.BlockSpec((bn, DH), lambda i: (i, 0)),
          pl.BlockSpec((DH, d_out), lambda i: (0, 0)),
      ],
      out_specs=pl.BlockSpec((bn, d_out), lambda i: (i, 0)),
      out_shape=jax.ShapeDtypeStruct((n, d_out), jnp.float32),
  )(x, w_out)


def kernel(h, edge_index, edge_weight, W_in, W_layers, W_out):
  n = h.shape[0]
  e = edge_weight.shape[0]
  depth = W_layers.shape[0]
  half = n // 2

  src = edge_index[0]
  dst = edge_index[1]
  w = edge_weight
  key = (dst >= half).astype(jnp.int32)
  perm = jnp.argsort(key, stable=True)
  ss = jnp.concatenate([src[perm], jnp.zeros((e,), jnp.int32)])
  dd = jnp.concatenate([dst[perm], jnp.zeros((e,), jnp.int32)])
  ww = jnp.concatenate([w[perm], jnp.zeros((e,), jnp.float32)])
  cnt_a = e - jnp.sum(key)

  unit = NS * K * G
  ec = ((e // 2 + e // 16) + unit - 1) // unit * unit
  tr = ec // NS // 128

  s_a, d_a, w_a = ss[:ec], dd[:ec], ww[:ec]
  m_a = d_a < half
  s_b = lax.dynamic_slice(ss, (cnt_a,), (ec,))
  d_b = lax.dynamic_slice(dd, (cnt_a,), (ec,))
  w_b = lax.dynamic_slice(ww, (cnt_a,), (ec,))
  m_b = d_b >= half

  srcP = jnp.stack([s_a, s_b]).reshape(2, NS, tr, 128)
  dstP = jnp.stack([
      jnp.where(m_a, d_a, 0),
      jnp.where(m_b, d_b - half, 0),
  ]).reshape(2, NS, tr, 128)
  wP = jnp.stack([
      jnp.where(m_a, w_a, 0.0),
      jnp.where(m_b, w_b, 0.0),
  ]).reshape(2, NS, tr, 128)

  zeros = jnp.zeros((half, 2, DHW), jnp.float32)
  diffuse = _make_diffusion(n, ec)

  x, xp = _mm_in(h, W_in)
  for l in range(depth):
    agg = diffuse(xp, srcP, dstP, wP, zeros).reshape(n, DH)
    x, xp = _mm_layer(agg, x, W_layers[l])
  return _mm_out(x, W_out)


# speedup vs baseline: 2.0255x; 2.0255x over previous
import functools

import jax
import jax.numpy as jnp
from jax import lax
from jax.experimental import pallas as pl
from jax.experimental.pallas import tpu as pltpu
from jax.experimental.pallas import tpu_sc as plsc

NC = 2
NS = 16
LANES = 16
K = 32
G = 32
R = 4
DH_HALF = 128


@functools.lru_cache(maxsize=None)
def _make_diffusion(n, ep):
  n_per_tile = ep // NS
  n_chunks = n_per_tile // K
  n_groups = n_chunks // G
  assert n_chunks % R == 0 and n_chunks % G == 0 and n_groups >= 2
  assert G >= 2 * R
  rows_per_tile = (n // NS) // 8 * 8
  rows_rem = n - NS * rows_per_tile
  mesh = plsc.VectorSubcoreMesh(core_axis_name="c", subcore_axis_name="s")

  @functools.partial(
      pl.kernel,
      mesh=mesh,
      out_type=jax.ShapeDtypeStruct((2 * n, DH_HALF), jnp.float32),
      scratch_types=(
          [
              pltpu.VMEM((2, G * K // 128, 128), jnp.int32),
              pltpu.VMEM((2, G * K // 128, 128), jnp.int32),
              pltpu.VMEM((2, G * K // 128, 128), jnp.float32),
          ]
          + [pltpu.VMEM((K, DH_HALF), jnp.float32) for _ in range(2 * R)]
          + [pltpu.VMEM((K,), jnp.int32) for _ in range(R)]
          + [pltpu.VMEM_SHARED((n, DH_HALF), jnp.float32)]
          + [pltpu.SemaphoreType.DMA for _ in range(2 * R + 1)]
      ),
  )
  def diffuse(x_hbm, src4, dst3, w3, z_hbm, out_hbm,
              sidx, didx, wts, *rest):
    rows_g = rest[0:R]
    rows_s = rest[R:2 * R]
    dbuf = rest[2 * R:3 * R]
    acc = rest[3 * R]
    semg = rest[3 * R + 1:4 * R + 1]
    sems = rest[4 * R + 1:5 * R + 1]
    semt = rest[5 * R + 1]
    c = lax.axis_index("c")
    s = lax.axis_index("s")
    cpr = 128 // K

    grows = G * K // 128

    def issue_tables(grp, slot):
      sl = pl.ds(grp * grows, grows)
      pltpu.async_copy(src4.at[c, s, sl], sidx.at[slot], semt)
      pltpu.async_copy(dst3.at[s, sl], didx.at[slot], semt)
      pltpu.async_copy(w3.at[s, sl], wts.at[slot], semt)

    def wait_tables(grp, slot):
      sl = pl.ds(grp * grows, grows)
      pltpu.make_async_copy(src4.at[c, s, sl], sidx.at[slot], semt).wait()
      pltpu.make_async_copy(dst3.at[s, sl], didx.at[slot], semt).wait()
      pltpu.make_async_copy(w3.at[s, sl], wts.at[slot], semt).wait()

    r0 = s * rows_per_tile
    pltpu.sync_copy(z_hbm.at[pl.ds(r0, rows_per_tile)],
                    acc.at[pl.ds(r0, rows_per_tile)])
    if rows_rem:
      @pl.when(s == NS - 1)
      def _zero_tail():
        rt = NS * rows_per_tile
        pltpu.sync_copy(z_hbm.at[pl.ds(rt, rows_rem)],
                        acc.at[pl.ds(rt, rows_rem)])

    issue_tables(0, 0)
    wait_tables(0, 0)
    plsc.subcore_barrier()

    def gidx(t):
      slot = (t // G) % 2
      tt = t % G
      return sidx.at[slot, tt // cpr, pl.ds((tt % cpr) * K, K)]

    for b in range(R):
      pltpu.async_copy(x_hbm.at[gidx(b)], rows_g[b], semg[b])

    def ring_body(p, carry):
      for b in range(R):
        t = p * R + b
        slot = (t // G) % 2
        tt = t % G
        pltpu.make_async_copy(x_hbm.at[gidx(t)], rows_g[b], semg[b]).wait()
        @pl.when(p > 0)
        def _drain():
          pltpu.make_async_copy(rows_s[b], acc.at[dbuf[b]], sems[b]).wait()

        for j in range(K // LANES):
          dbuf[b][pl.ds(j * LANES, LANES)] = (
              didx[slot, tt // cpr, pl.ds((tt % cpr) * K + j * LANES, LANES)])

        def edge_body(g16, c2):
          w16 = wts[slot, tt // cpr,
                    pl.ds((tt % cpr) * K + g16 * LANES, LANES)]
          for l in range(LANES):
            i = g16 * LANES + l
            wb = w16[l]
            for j in range(DH_HALF // LANES):
              fsl = pl.ds(j * LANES, LANES)
              rows_s[b][i, fsl] = rows_g[b][i, fsl] * wb
          return c2

        lax.fori_loop(0, K // LANES, edge_body, 0)

        pltpu.async_copy(rows_s[b], acc.at[dbuf[b]], sems[b], add=True)

        @pl.when((tt == R - 1) & (t < (n_groups - 1) * G))
        def _refill():
          issue_tables(t // G + 1, 1 - slot)

        t2 = t + R
        @pl.when((t2 % G == 0) & (t2 < n_chunks))
        def _tables_ready():
          wait_tables(t2 // G, (t2 // G) % 2)

        @pl.when(t2 < n_chunks)
        def _prefetch():
          pltpu.async_copy(x_hbm.at[gidx(t2)], rows_g[b], semg[b])
      return carry

    lax.fori_loop(0, n_chunks // R, ring_body, 0)
    for b in range(R):
      pltpu.make_async_copy(rows_s[b], acc.at[dbuf[b]], sems[b]).wait()
    plsc.subcore_barrier()

    pltpu.sync_copy(acc.at[pl.ds(r0, rows_per_tile)],
                    out_hbm.at[pl.ds(c * n + r0, rows_per_tile)])
    if rows_rem:
      @pl.when(s == NS - 1)
      def _out_tail():
        rt = NS * rows_per_tile
        pltpu.sync_copy(acc.at[pl.ds(rt, rows_rem)],
                        out_hbm.at[pl.ds(c * n + rt, rows_rem)])

  return diffuse


def _mm_in(h, w_in):
  n, d_in = h.shape
  bn = 1000

  def body(h_ref, w_ref, o_ref):
    y = jnp.dot(h_ref[...], w_ref[...], preferred_element_type=jnp.float32)
    o_ref[0] = y[:, :DH_HALF]
    o_ref[1] = y[:, DH_HALF:]

  return pl.pallas_call(
      body,
      grid=(n // bn,),
      in_specs=[
          pl.BlockSpec((bn, d_in), lambda i: (i, 0)),
          pl.BlockSpec((d_in, 2 * DH_HALF), lambda i: (0, 0)),
      ],
      out_specs=pl.BlockSpec((2, bn, DH_HALF), lambda i: (0, i, 0)),
      out_shape=jax.ShapeDtypeStruct((2, n, DH_HALF), jnp.float32),
  )(h, w_in)


def _mm_layer(agg2, x2, w):
  n = x2.shape[1]
  bn = 1000

  def body(a_ref, x_ref, w_ref, o_ref):
    a = jnp.concatenate([a_ref[0], a_ref[1]], axis=1)
    x = jnp.concatenate([x_ref[0], x_ref[1]], axis=1)
    y = jnp.dot(a, w_ref[...], preferred_element_type=jnp.float32)
    y = y * jax.nn.sigmoid(y) + x
    o_ref[0] = y[:, :DH_HALF]
    o_ref[1] = y[:, DH_HALF:]

  spec2 = pl.BlockSpec((2, bn, DH_HALF), lambda i: (0, i, 0))
  return pl.pallas_call(
      body,
      grid=(n // bn,),
      in_specs=[spec2, spec2,
                pl.BlockSpec((2 * DH_HALF, 2 * DH_HALF), lambda i: (0, 0))],
      out_specs=spec2,
      out_shape=jax.ShapeDtypeStruct((2, n, DH_HALF), jnp.float32),
  )(agg2, x2, w)


def _mm_out(x2, w_out):
  n = x2.shape[1]
  d_out = w_out.shape[1]
  bn = 1000

  def body(x_ref, w_ref, o_ref):
    x = jnp.concatenate([x_ref[0], x_ref[1]], axis=1)
    o_ref[...] = jnp.dot(x, w_ref[...], preferred_element_type=jnp.float32)

  return pl.pallas_call(
      body,
      grid=(n // bn,),
      in_specs=[
          pl.BlockSpec((2, bn, DH_HALF), lambda i: (0, i, 0)),
          pl.BlockSpec((2 * DH_HALF, d_out), lambda i: (0, 0)),
      ],
      out_specs=pl.BlockSpec((bn, d_out), lambda i: (i, 0)),
      out_shape=jax.ShapeDtypeStruct((n, d_out), jnp.float32),
  )(x2, w_out)


def kernel(h, edge_index, edge_weight, W_in, W_layers, W_out):
  n = h.shape[0]
  e = edge_weight.shape[0]
  depth = W_layers.shape[0]

  unit = NS * K * G
  ep = ((e + unit - 1) // unit) * unit
  src = edge_index[0]
  dst = edge_index[1]
  w = edge_weight
  if ep != e:
    pad = ep - e
    src = jnp.concatenate([src, jnp.zeros((pad,), jnp.int32)])
    dst = jnp.concatenate([dst, jnp.zeros((pad,), jnp.int32)])
    w = jnp.concatenate([w, jnp.zeros((pad,), jnp.float32)])

  src3 = src.reshape(NS, -1, 128)
  src4 = jnp.stack([src3, src3 + n])
  dst3 = dst.reshape(NS, -1, 128)
  w3 = w.reshape(NS, -1, 128)
  zeros = jnp.zeros((n, DH_HALF), jnp.float32)
  diffuse = _make_diffusion(n, ep)

  x2 = _mm_in(h, W_in)
  for l in range(depth):
    agg = diffuse(x2.reshape(2 * n, DH_HALF), src4, dst3, w3, zeros)
    x2 = _mm_layer(agg.reshape(2, n, DH_HALF), x2, W_layers[l])
  return _mm_out(x2, W_out)
